# baseline (speedup 1.0000x reference)
import jax
import jax.numpy as jnp
from jax.experimental import pallas as pl
from jax.experimental.pallas import tpu as pltpu

_H = 32
_P = 8
_T_ENC = 16
_PACK = 4
_SPLIT = 60000
_BLOCK_N = 20000
_BLOCK_W = 5000


def _const_part(t_ref, pa_ref, wt1_ref, bt1_ref, wt2_ref, bt2_ref,
                wd1, bd1_ref):
    tv = jnp.dot(
        jnp.maximum(jnp.dot(t_ref[...], wt1_ref[...],
                            preferred_element_type=jnp.float32)
                    + bt1_ref[...], 0.0),
        wt2_ref[...], preferred_element_type=jnp.float32) + bt2_ref[...]
    return (jnp.dot(pa_ref[...], wd1[_H:_H + _P, :],
                    preferred_element_type=jnp.float32)
            + jnp.dot(tv, wd1[_H + _P:, :], preferred_element_type=jnp.float32)
            + bd1_ref[...])


def _body_narrow(t_ref, pa_ref, wt1_ref, bt1_ref, wt2_ref, bt2_ref,
                 wd1_ref, bd1_ref, wd2_ref, bd2_ref, wd3_ref, bd3_ref,
                 ze_ref, out_ref, const_ref):
    @pl.when(pl.program_id(0) == 0)
    def _():
        const_ref[...] = _const_part(t_ref, pa_ref, wt1_ref, bt1_ref,
                                     wt2_ref, bt2_ref, wd1_ref[...], bd1_ref)

    h = jnp.maximum(
        jnp.dot(ze_ref[...], wd1_ref[:_H, :], preferred_element_type=jnp.float32)
        + const_ref[...], 0.0)
    h = jnp.maximum(
        jnp.dot(h, wd2_ref[...], preferred_element_type=jnp.float32)
        + bd2_ref[...], 0.0)
    out_ref[...] = (jnp.dot(h, wd3_ref[...], preferred_element_type=jnp.float32)
                    + bd3_ref[...])


def _body_wide(t_ref, pa_ref, wt1_ref, bt1_ref, wt2_ref, bt2_ref,
               wd1_ref, bd1_ref, wd2_ref, bd2_ref, wd3_ref, bd3_ref,
               ze_ref, out_ref,
               w1p_ref, w2p_ref, w3p_ref, const_ref, b2p_ref, b3p_ref):
    @pl.when(pl.program_id(0) == 0)
    def _():
        wd1 = wd1_ref[...]
        const = _const_part(t_ref, pa_ref, wt1_ref, bt1_ref,
                            wt2_ref, bt2_ref, wd1, bd1_ref)
        const_ref[...] = jnp.concatenate([const] * _PACK, axis=1)
        b2p_ref[...] = jnp.concatenate([bd2_ref[...]] * _PACK, axis=1)
        b3p_ref[...] = jnp.concatenate([bd3_ref[...]] * _PACK, axis=1)
        a1 = wd1[:_H, :]
        a2 = wd2_ref[...]
        a3 = wd3_ref[...]
        w1p_ref[...] = jnp.zeros_like(w1p_ref)
        w2p_ref[...] = jnp.zeros_like(w2p_ref)
        w3p_ref[...] = jnp.zeros_like(w3p_ref)
        for i in range(_PACK):
            w1p_ref[_H * i:_H * (i + 1), 2 * _H * i:2 * _H * (i + 1)] = a1
            w2p_ref[2 * _H * i:2 * _H * (i + 1), _H * i:_H * (i + 1)] = a2
            w3p_ref[_H * i:_H * (i + 1), _H * i:_H * (i + 1)] = a3

    h = jnp.maximum(
        jnp.dot(ze_ref[...], w1p_ref[...], preferred_element_type=jnp.float32)
        + const_ref[...], 0.0)
    h = jnp.maximum(
        jnp.dot(h, w2p_ref[...], preferred_element_type=jnp.float32)
        + b2p_ref[...], 0.0)
    out_ref[...] = (jnp.dot(h, w3p_ref[...], preferred_element_type=jnp.float32)
                    + b3p_ref[...])


def _full(shape):
    return pl.BlockSpec(shape, lambda i: (0,) * len(shape))


def _weight_args(t, person_attrs, Wt1, bt1, Wt2, bt2, Wd1, bd1, Wd2, bd2,
                 Wd3, bd3):
    specs = [
        _full((1, 1)), _full((1, _P)),
        _full(Wt1.shape), _full((1, _T_ENC)),
        _full(Wt2.shape), _full((1, _T_ENC)),
        _full(Wd1.shape), _full((1, 2 * _H)),
        _full(Wd2.shape), _full((1, _H)),
        _full(Wd3.shape), _full((1, _H)),
    ]
    args = (
        jnp.reshape(t, (1, 1)), jnp.reshape(person_attrs, (1, _P)),
        Wt1, jnp.reshape(bt1, (1, _T_ENC)),
        Wt2, jnp.reshape(bt2, (1, _T_ENC)),
        Wd1, jnp.reshape(bd1, (1, 2 * _H)),
        Wd2, jnp.reshape(bd2, (1, _H)),
        Wd3, jnp.reshape(bd3, (1, _H)),
    )
    return specs, args


def kernel(t, zone_embedding, zone_features, edge_index, person_attrs,
           W1, b1, W2, b2, Wt1, bt1, Wt2, bt2,
           Wd1, bd1, Wd2, bd2, Wd3, bd3):
    del zone_features, edge_index, W1, b1, W2, b2
    n = zone_embedding.shape[0]
    specs, args = _weight_args(t, person_attrs, Wt1, bt1, Wt2, bt2,
                               Wd1, bd1, Wd2, bd2, Wd3, bd3)

    m = n - _SPLIT
    m4 = m // _PACK
    ze_w = jnp.reshape(zone_embedding[_SPLIT:], (m4, _PACK * _H))
    out_w = pl.pallas_call(
        _body_wide,
        grid=(m4 // _BLOCK_W,),
        in_specs=specs + [pl.BlockSpec((_BLOCK_W, _PACK * _H),
                                       lambda i: (i, 0))],
        out_specs=pl.BlockSpec((_BLOCK_W, _PACK * _H), lambda i: (i, 0)),
        out_shape=jax.ShapeDtypeStruct((m4, _PACK * _H), jnp.float32),
        scratch_shapes=[
            pltpu.VMEM((_PACK * _H, _PACK * 2 * _H), jnp.float32),
            pltpu.VMEM((_PACK * 2 * _H, _PACK * _H), jnp.float32),
            pltpu.VMEM((_PACK * _H, _PACK * _H), jnp.float32),
            pltpu.VMEM((1, _PACK * 2 * _H), jnp.float32),
            pltpu.VMEM((1, _PACK * _H), jnp.float32),
            pltpu.VMEM((1, _PACK * _H), jnp.float32),
        ],
    )(*args, ze_w)

    out_n = pl.pallas_call(
        _body_narrow,
        grid=(_SPLIT // _BLOCK_N,),
        in_specs=specs + [pl.BlockSpec((_BLOCK_N, _H), lambda i: (i, 0))],
        out_specs=pl.BlockSpec((_BLOCK_N, _H), lambda i: (i, 0)),
        out_shape=jax.ShapeDtypeStruct((_SPLIT, _H), jnp.float32),
        scratch_shapes=[pltpu.VMEM((1, 2 * _H), jnp.float32)],
    )(*args, zone_embedding[:_SPLIT])

    return jnp.concatenate([out_n, jnp.reshape(out_w, (m, _H))], axis=0)


# speedup vs baseline: 1.7813x; 1.7813x over previous
import jax
import jax.numpy as jnp
from jax.experimental import pallas as pl
from jax.experimental.pallas import tpu as pltpu

_H = 32
_P = 8
_T_ENC = 16
_BLOCK = 20000


def _body(t_ref, pa_ref, wt1_ref, bt1_ref, wt2_ref, bt2_ref,
          wd1_ref, bd1_ref, wd2_ref, bd2_ref, wd3_ref, bd3_ref,
          ze_ref, out_ref, const_ref):
    @pl.when(pl.program_id(0) == 0)
    def _():
        tv = jnp.dot(
            jnp.maximum(jnp.dot(t_ref[...], wt1_ref[...],
                                preferred_element_type=jnp.float32)
                        + bt1_ref[...], 0.0),
            wt2_ref[...], preferred_element_type=jnp.float32) + bt2_ref[...]
        wd1 = wd1_ref[...]
        const_ref[...] = (
            jnp.dot(pa_ref[...], wd1[_H:_H + _P, :],
                    preferred_element_type=jnp.float32)
            + jnp.dot(tv, wd1[_H + _P:, :], preferred_element_type=jnp.float32)
            + bd1_ref[...])

    z = ze_ref[...].astype(jnp.float32)
    h = jnp.maximum(
        jnp.dot(z, wd1_ref[:_H, :], preferred_element_type=jnp.float32)
        + const_ref[...], 0.0)
    h = jnp.maximum(
        jnp.dot(h, wd2_ref[...], preferred_element_type=jnp.float32)
        + bd2_ref[...], 0.0)
    v = (jnp.dot(h, wd3_ref[...], preferred_element_type=jnp.float32)
         + bd3_ref[...])
    out_ref[...] = v.astype(jnp.bfloat16)


def kernel(t, zone_embedding, zone_features, edge_index, person_attrs,
           W1, b1, W2, b2, Wt1, bt1, Wt2, bt2,
           Wd1, bd1, Wd2, bd2, Wd3, bd3):
    del zone_features, edge_index, W1, b1, W2, b2
    n = zone_embedding.shape[0]
    grid = (n // _BLOCK,)

    def full(shape):
        return pl.BlockSpec(shape, lambda i: (0,) * len(shape))

    out = pl.pallas_call(
        _body,
        grid=grid,
        in_specs=[
            full((1, 1)),
            full((1, _P)),
            full(Wt1.shape),
            full((1, _T_ENC)),
            full(Wt2.shape),
            full((1, _T_ENC)),
            full(Wd1.shape),
            full((1, 2 * _H)),
            full(Wd2.shape),
            full((1, _H)),
            full(Wd3.shape),
            full((1, _H)),
            pl.BlockSpec((_BLOCK, _H), lambda i: (i, 0)),
        ],
        out_specs=pl.BlockSpec((_BLOCK, _H), lambda i: (i, 0)),
        out_shape=jax.ShapeDtypeStruct((n, _H), jnp.bfloat16),
        scratch_shapes=[pltpu.VMEM((1, 2 * _H), jnp.float32)],
    )(
        jnp.reshape(t, (1, 1)),
        jnp.reshape(person_attrs, (1, _P)),
        Wt1,
        jnp.reshape(bt1, (1, _T_ENC)),
        Wt2,
        jnp.reshape(bt2, (1, _T_ENC)),
        Wd1,
        jnp.reshape(bd1, (1, 2 * _H)),
        Wd2,
        jnp.reshape(bd2, (1, _H)),
        Wd3,
        jnp.reshape(bd3, (1, _H)),
        zone_embedding.astype(jnp.bfloat16),
    )
    return out.astype(jnp.float32)
